# baseline (speedup 1.0000x reference)
import functools

import jax
import jax.numpy as jnp
from jax import lax
from jax.experimental import pallas as pl
from jax.experimental.pallas import tpu as pltpu
from jax.experimental.pallas import tpu_sc as plsc

NC = 2
NS = 16
NW = NC * NS
CH = 128


def _agg_call(NPAD, F, NCH):
    mesh = plsc.VectorSubcoreMesh(core_axis_name="c", subcore_axis_name="s", num_cores=NC, num_subcores=NS)
    stripe = NPAD // NS
    HNCH = NCH // 2
    HPAIR = HNCH // 2

    @functools.partial(
        pl.kernel,
        out_type=jax.ShapeDtypeStruct((NC * NPAD, F), jnp.float32),
        mesh=mesh,
        scratch_types=[
            pltpu.VMEM((HNCH, CH), jnp.int32),
            pltpu.VMEM((HNCH, CH), jnp.int32),
            pltpu.VMEM((CH, F), jnp.float32),
            pltpu.VMEM((CH, F), jnp.float32),
            pltpu.VMEM_SHARED((NPAD, F), jnp.float32),
            pltpu.SemaphoreType.DMA,
            pltpu.SemaphoreType.DMA,
        ],
    )
    def k(src_hbm, dst_hbm, s_hbm, zeros_hbm, out_hbm, idx_s, idx_d,
          rows0, rows1, acc_sh, sem0, sem1):
        c = lax.axis_index("c")
        s = lax.axis_index("s")
        wid = c * NS + s
        row0 = s * stripe
        pltpu.sync_copy(zeros_hbm.at[pl.ds(row0, stripe)],
                        acc_sh.at[pl.ds(row0, stripe)])
        plsc.subcore_barrier()

        for h in range(2):
            base = wid * NCH + h * HNCH
            pltpu.sync_copy(src_hbm.at[pl.ds(base, HNCH)], idx_s)
            pltpu.sync_copy(dst_hbm.at[pl.ds(base, HNCH)], idx_d)
            pltpu.async_copy(s_hbm.at[idx_s.at[0]], rows0, sem0)

            def body(j, carry):
                k0 = 2 * j
                pltpu.async_copy(s_hbm.at[idx_s.at[k0 + 1]], rows1, sem1)
                pltpu.make_async_copy(s_hbm.at[idx_s.at[k0]], rows0,
                                      sem0).wait()
                pltpu.sync_copy(rows0, acc_sh.at[idx_d.at[k0]], add=True)

                @pl.when(k0 + 2 < HNCH)
                def _():
                    pltpu.async_copy(s_hbm.at[idx_s.at[k0 + 2]], rows0, sem0)

                pltpu.make_async_copy(s_hbm.at[idx_s.at[k0 + 1]], rows1,
                                      sem1).wait()
                pltpu.sync_copy(rows1, acc_sh.at[idx_d.at[k0 + 1]], add=True)
                return carry

            lax.fori_loop(0, HPAIR, body, 0)

        plsc.subcore_barrier()
        pltpu.sync_copy(acc_sh.at[pl.ds(row0, stripe)],
                        out_hbm.at[pl.ds(c * NPAD + row0, stripe)])

    return k


def _deg_call(NPAD, F, NCH):
    mesh = plsc.VectorSubcoreMesh(core_axis_name="c", subcore_axis_name="s", num_cores=NC, num_subcores=NS)
    stripe = NPAD // NS

    @functools.partial(
        pl.kernel,
        out_type=jax.ShapeDtypeStruct((NC * NPAD, F), jnp.float32),
        mesh=mesh,
        scratch_types=[
            pltpu.VMEM((NCH, CH), jnp.int32),
            pltpu.VMEM((CH, F), jnp.float32),
            pltpu.VMEM_SHARED((NPAD, F), jnp.float32),
        ],
    )
    def k(dst_hbm, zeros_hbm, ones_hbm, out_hbm, idx_d, ones_v, acc_sh):
        c = lax.axis_index("c")
        s = lax.axis_index("s")
        wid = c * NS + s
        row0 = s * stripe
        pltpu.sync_copy(dst_hbm.at[pl.ds(wid * NCH, NCH)], idx_d)
        pltpu.sync_copy(zeros_hbm.at[pl.ds(row0, stripe)],
                        acc_sh.at[pl.ds(row0, stripe)])
        pltpu.sync_copy(ones_hbm, ones_v)
        plsc.subcore_barrier()

        def body(kk, carry):
            pltpu.sync_copy(ones_v, acc_sh.at[idx_d.at[kk]], add=True)
            return carry

        lax.fori_loop(0, NCH, body, 0)
        plsc.subcore_barrier()
        pltpu.sync_copy(acc_sh.at[pl.ds(row0, stripe)],
                        out_hbm.at[pl.ds(c * NPAD + row0, stripe)])

    return k


def _first_call(NPAD, Fin, F):
    def body(x_ref, w_ref, degc_ref, s_ref, dinv_ref):
        deg = (degc_ref[0:NPAD, 0:1] + degc_ref[NPAD:2 * NPAD, 0:1]) + 1.0
        dinv = lax.rsqrt(jnp.maximum(deg, 1.0))
        dinv_ref[...] = dinv
        u = jnp.dot(x_ref[...], w_ref[...], preferred_element_type=jnp.float32)
        s_ref[...] = u * dinv

    return pl.pallas_call(
        body,
        out_shape=(
            jax.ShapeDtypeStruct((NPAD, F), jnp.float32),
            jax.ShapeDtypeStruct((NPAD, 1), jnp.float32),
        ),
    )


def _mid_call(NPAD, F):
    def body(agg_ref, sp_ref, dinv_ref, b_ref, w_ref, out_ref):
        a = agg_ref[0:NPAD, :] + agg_ref[NPAD:2 * NPAD, :] + sp_ref[...]
        dinv = dinv_ref[...]
        h = jnp.maximum(dinv * a + b_ref[...], 0.0)
        out_ref[...] = jnp.dot(h, w_ref[...],
                               preferred_element_type=jnp.float32) * dinv

    return pl.pallas_call(
        body, out_shape=jax.ShapeDtypeStruct((NPAD, F), jnp.float32))


def _pool_call(NPAD, F, NCLS):
    def body(agg_ref, sp_ref, dinv_ref, b_ref, batch_ref, lw_ref, lb_ref,
             out_ref):
        a = agg_ref[0:NPAD, :] + agg_ref[NPAD:2 * NPAD, :] + sp_ref[...]
        h = jnp.maximum(dinv_ref[...] * a + b_ref[...], 0.0)
        gid = batch_ref[...]
        onehot = (gid == lax.broadcasted_iota(jnp.int32, (NPAD, 128), 1)
                  ).astype(jnp.float32)
        sums = lax.dot_general(onehot, h, (((0,), (0,)), ((), ())),
                               preferred_element_type=jnp.float32)
        cnt = lax.dot_general(onehot, jnp.ones((NPAD, 8), jnp.float32),
                              (((0,), (0,)), ((), ())),
                              preferred_element_type=jnp.float32)
        mean = sums / jnp.maximum(cnt[:, 0:1], 1.0)
        out_ref[...] = jnp.dot(mean, lw_ref[...],
                               preferred_element_type=jnp.float32) + lb_ref[...]

    return pl.pallas_call(
        body, out_shape=jax.ShapeDtypeStruct((128, NCLS), jnp.float32))


def kernel(x, edge_index, batch, W0, W1, W2, W3, W4, b0, b1, b2, b3, b4,
           lin_w, lin_b):
    N, Fin = x.shape
    F = W0.shape[1]
    E = edge_index.shape[1]
    NCLS = lin_w.shape[1]
    G = 64

    NPAD = ((N + 255) // 256) * 256
    NCH = -(-E // (NW * CH))
    NCH = NCH + (NCH % 2)
    EPAD = NW * CH * NCH

    src = jnp.concatenate(
        [edge_index[0], jnp.full((EPAD - E,), N, jnp.int32)]
    ).reshape(NW * NCH, CH)
    dst = jnp.concatenate(
        [edge_index[1], jnp.full((EPAD - E,), N, jnp.int32)]
    ).reshape(NW * NCH, CH)
    xp = jnp.zeros((NPAD, Fin), jnp.float32).at[:N, :].set(x)
    batchp = jnp.full((NPAD, 1), G, jnp.int32).at[:N, 0].set(batch)
    zerosF = jnp.zeros((NPAD, F), jnp.float32)
    onesF = jnp.ones((CH, F), jnp.float32)
    Ws = [W0, W1, W2, W3, W4]
    bs = [b.reshape(1, F) for b in (b0, b1, b2, b3, b4)]

    agg = _agg_call(NPAD, F, NCH)
    degc = _deg_call(NPAD, F, NCH)(dst, zerosF, onesF)
    s, dinv = _first_call(NPAD, Fin, F)(xp, W0, degc)
    for i in range(1, 5):
        a = agg(src, dst, s, zerosF)
        s = _mid_call(NPAD, F)(a, s, dinv, bs[i - 1], Ws[i])
    a = agg(src, dst, s, zerosF)
    out = _pool_call(NPAD, F, NCLS)(a, s, dinv, bs[4], batchp,
                                    lin_w, lin_b.reshape(1, NCLS))
    return out[:G]


# speedup vs baseline: 3.1130x; 3.1130x over previous
import functools

import jax
import jax.numpy as jnp
from jax import lax
from jax.experimental import pallas as pl
from jax.experimental.pallas import tpu as pltpu
from jax.experimental.pallas import tpu_sc as plsc

NC = 2
NS = 16
NW = NC * NS
CH = 128


def _agg_call(NPAD, F, NCH):
    mesh = plsc.VectorSubcoreMesh(core_axis_name="c", subcore_axis_name="s", num_cores=NC, num_subcores=NS)
    stripe = NPAD // NS
    HNCH = NCH // 2
    HPAIR = HNCH // 2

    @functools.partial(
        pl.kernel,
        out_type=jax.ShapeDtypeStruct((NC * NPAD, F), jnp.float32),
        mesh=mesh,
        scratch_types=[
            pltpu.VMEM((HNCH, CH), jnp.int32),
            pltpu.VMEM((HNCH, CH), jnp.int32),
            pltpu.VMEM((CH, F), jnp.float32),
            pltpu.VMEM((CH, F), jnp.float32),
            pltpu.VMEM_SHARED((NPAD, F), jnp.float32),
            pltpu.SemaphoreType.DMA,
            pltpu.SemaphoreType.DMA,
        ],
    )
    def k(src_hbm, dst_hbm, s_hbm, zeros_hbm, out_hbm, idx_s, idx_d,
          rows0, rows1, acc_sh, sem0, sem1):
        c = lax.axis_index("c")
        s = lax.axis_index("s")
        wid = c * NS + s
        row0 = s * stripe
        pltpu.sync_copy(zeros_hbm.at[pl.ds(row0, stripe)],
                        acc_sh.at[pl.ds(row0, stripe)])
        plsc.subcore_barrier()

        for h in range(2):
            base = wid * NCH + h * HNCH
            pltpu.sync_copy(src_hbm.at[pl.ds(base, HNCH)], idx_s)
            pltpu.sync_copy(dst_hbm.at[pl.ds(base, HNCH)], idx_d)
            pltpu.async_copy(s_hbm.at[idx_s.at[0]], rows0, sem0)

            def body(j, carry):
                k0 = 2 * j
                pltpu.async_copy(s_hbm.at[idx_s.at[k0 + 1]], rows1, sem1)
                pltpu.make_async_copy(s_hbm.at[idx_s.at[k0]], rows0,
                                      sem0).wait()
                pltpu.sync_copy(rows0, acc_sh.at[idx_d.at[k0]], add=True)

                @pl.when(k0 + 2 < HNCH)
                def _():
                    pltpu.async_copy(s_hbm.at[idx_s.at[k0 + 2]], rows0, sem0)

                pltpu.make_async_copy(s_hbm.at[idx_s.at[k0 + 1]], rows1,
                                      sem1).wait()
                pltpu.sync_copy(rows1, acc_sh.at[idx_d.at[k0 + 1]], add=True)
                return carry

            lax.fori_loop(0, HPAIR, body, 0)

        plsc.subcore_barrier()
        pltpu.sync_copy(acc_sh.at[pl.ds(row0, stripe)],
                        out_hbm.at[pl.ds(c * NPAD + row0, stripe)])

    return k


def _deg_call(NPAD, F, NCH):
    mesh = plsc.VectorSubcoreMesh(core_axis_name="c", subcore_axis_name="s", num_cores=NC, num_subcores=NS)
    stripe = NPAD // NS

    @functools.partial(
        pl.kernel,
        out_type=jax.ShapeDtypeStruct((NC * NPAD, F), jnp.float32),
        mesh=mesh,
        scratch_types=[
            pltpu.VMEM((NCH, CH), jnp.int32),
            pltpu.VMEM((CH, F), jnp.float32),
            pltpu.VMEM_SHARED((NPAD, F), jnp.float32),
        ],
    )
    def k(dst_hbm, zeros_hbm, ones_hbm, out_hbm, idx_d, ones_v, acc_sh):
        c = lax.axis_index("c")
        s = lax.axis_index("s")
        wid = c * NS + s
        row0 = s * stripe
        pltpu.sync_copy(dst_hbm.at[pl.ds(wid * NCH, NCH)], idx_d)
        pltpu.sync_copy(zeros_hbm.at[pl.ds(row0, stripe)],
                        acc_sh.at[pl.ds(row0, stripe)])
        pltpu.sync_copy(ones_hbm, ones_v)
        plsc.subcore_barrier()

        def body(kk, carry):
            pltpu.sync_copy(ones_v, acc_sh.at[idx_d.at[kk]], add=True)
            return carry

        lax.fori_loop(0, NCH, body, 0)
        plsc.subcore_barrier()
        pltpu.sync_copy(acc_sh.at[pl.ds(row0, stripe)],
                        out_hbm.at[pl.ds(c * NPAD + row0, stripe)])

    return k


def _first_call(NPAD, Fin, F):
    def body(x_ref, w_ref, degc_ref, s_ref, dinv_ref):
        deg = (degc_ref[0:NPAD, 0:1] + degc_ref[NPAD:2 * NPAD, 0:1]) + 1.0
        dinv = lax.rsqrt(jnp.maximum(deg, 1.0))
        dinv_ref[...] = dinv
        u = jnp.dot(x_ref[...], w_ref[...], preferred_element_type=jnp.float32)
        s_ref[...] = u * dinv

    return pl.pallas_call(
        body,
        out_shape=(
            jax.ShapeDtypeStruct((NPAD, F), jnp.float32),
            jax.ShapeDtypeStruct((NPAD, 1), jnp.float32),
        ),
    )


def _mid_call(NPAD, F):
    def body(agg_ref, sp_ref, dinv_ref, b_ref, w_ref, out_ref):
        a = agg_ref[0:NPAD, :] + agg_ref[NPAD:2 * NPAD, :] + sp_ref[...]
        dinv = dinv_ref[...]
        h = jnp.maximum(dinv * a + b_ref[...], 0.0)
        out_ref[...] = jnp.dot(h, w_ref[...],
                               preferred_element_type=jnp.float32) * dinv

    return pl.pallas_call(
        body, out_shape=jax.ShapeDtypeStruct((NPAD, F), jnp.float32))


def _pool_call(NPAD, F, NCLS):
    def body(agg_ref, sp_ref, dinv_ref, b_ref, batch_ref, lw_ref, lb_ref,
             out_ref):
        a = agg_ref[0:NPAD, :] + agg_ref[NPAD:2 * NPAD, :] + sp_ref[...]
        h = jnp.maximum(dinv_ref[...] * a + b_ref[...], 0.0)
        gid = batch_ref[...]
        onehot = (gid == lax.broadcasted_iota(jnp.int32, (NPAD, 128), 1)
                  ).astype(jnp.float32)
        sums = lax.dot_general(onehot, h, (((0,), (0,)), ((), ())),
                               preferred_element_type=jnp.float32)
        cnt = lax.dot_general(onehot, jnp.ones((NPAD, 8), jnp.float32),
                              (((0,), (0,)), ((), ())),
                              preferred_element_type=jnp.float32)
        mean = sums / jnp.maximum(cnt[:, 0:1], 1.0)
        out_ref[...] = jnp.dot(mean, lw_ref[...],
                               preferred_element_type=jnp.float32) + lb_ref[...]

    return pl.pallas_call(
        body, out_shape=jax.ShapeDtypeStruct((128, NCLS), jnp.float32))


def kernel(x, edge_index, batch, W0, W1, W2, W3, W4, b0, b1, b2, b3, b4,
           lin_w, lin_b):
    N, Fin = x.shape
    F = W0.shape[1]
    E = edge_index.shape[1]
    NCLS = lin_w.shape[1]
    G = 64

    NPAD = ((N + 255) // 256) * 256
    NCH = -(-E // (NW * CH))
    NCH = NCH + (NCH % 2)
    EPAD = NW * CH * NCH

    pad_idx = (N + jnp.arange(EPAD - E, dtype=jnp.int32) % (NPAD - N))
    src = jnp.concatenate([edge_index[0], pad_idx]).reshape(NW * NCH, CH)
    dst = jnp.concatenate([edge_index[1], pad_idx]).reshape(NW * NCH, CH)
    xp = jnp.zeros((NPAD, Fin), jnp.float32).at[:N, :].set(x)
    batchp = jnp.full((NPAD, 1), G, jnp.int32).at[:N, 0].set(batch)
    zerosF = jnp.zeros((NPAD, F), jnp.float32)
    onesF = jnp.ones((CH, F), jnp.float32)
    Ws = [W0, W1, W2, W3, W4]
    bs = [b.reshape(1, F) for b in (b0, b1, b2, b3, b4)]

    agg = _agg_call(NPAD, F, NCH)
    degc = _deg_call(NPAD, F, NCH)(dst, zerosF, onesF)
    s, dinv = _first_call(NPAD, Fin, F)(xp, W0, degc)
    for i in range(1, 5):
        a = agg(src, dst, s, zerosF)
        s = _mid_call(NPAD, F)(a, s, dinv, bs[i - 1], Ws[i])
    a = agg(src, dst, s, zerosF)
    out = _pool_call(NPAD, F, NCLS)(a, s, dinv, bs[4], batchp,
                                    lin_w, lin_b.reshape(1, NCLS))
    return out[:G]


# speedup vs baseline: 3.1182x; 1.0017x over previous
import functools

import jax
import jax.numpy as jnp
from jax import lax
from jax.experimental import pallas as pl
from jax.experimental.pallas import tpu as pltpu
from jax.experimental.pallas import tpu_sc as plsc

NC = 2
NS = 16
NW = NC * NS
CH = 128


def _agg_call(NPAD, F, NCH):
    mesh = plsc.VectorSubcoreMesh(core_axis_name="c", subcore_axis_name="s", num_cores=NC, num_subcores=NS)
    stripe = NPAD // NS
    HNCH = NCH // 2
    HPAIR = HNCH // 2

    @functools.partial(
        pl.kernel,
        out_type=jax.ShapeDtypeStruct((NC * NPAD, F), jnp.float32),
        mesh=mesh,
        scratch_types=[
            pltpu.VMEM((HNCH, CH), jnp.int32),
            pltpu.VMEM((HNCH, CH), jnp.int32),
            pltpu.VMEM((CH, F), jnp.float32),
            pltpu.VMEM((CH, F), jnp.float32),
            pltpu.VMEM_SHARED((NPAD, F), jnp.float32),
            pltpu.SemaphoreType.DMA,
            pltpu.SemaphoreType.DMA,
        ],
    )
    def k(src_hbm, dst_hbm, s_hbm, zeros_hbm, out_hbm, idx_s, idx_d,
          rows0, rows1, acc_sh, sem0, sem1):
        c = lax.axis_index("c")
        s = lax.axis_index("s")
        wid = c * NS + s
        row0 = s * stripe
        pltpu.sync_copy(zeros_hbm.at[pl.ds(row0, stripe)],
                        acc_sh.at[pl.ds(row0, stripe)])
        plsc.subcore_barrier()

        for h in range(2):
            base = wid * NCH + h * HNCH
            pltpu.sync_copy(src_hbm.at[pl.ds(base, HNCH)], idx_s)
            pltpu.sync_copy(dst_hbm.at[pl.ds(base, HNCH)], idx_d)
            pltpu.async_copy(s_hbm.at[idx_s.at[0]], rows0, sem0)

            def body(j, carry):
                k0 = 2 * j
                pltpu.async_copy(s_hbm.at[idx_s.at[k0 + 1]], rows1, sem1)
                pltpu.make_async_copy(s_hbm.at[idx_s.at[k0]], rows0,
                                      sem0).wait()
                pltpu.sync_copy(rows0, acc_sh.at[idx_d.at[k0]], add=True)

                @pl.when(k0 + 2 < HNCH)
                def _():
                    pltpu.async_copy(s_hbm.at[idx_s.at[k0 + 2]], rows0, sem0)

                pltpu.make_async_copy(s_hbm.at[idx_s.at[k0 + 1]], rows1,
                                      sem1).wait()
                pltpu.sync_copy(rows1, acc_sh.at[idx_d.at[k0 + 1]], add=True)
                return carry

            lax.fori_loop(0, HPAIR, body, 0)

        plsc.subcore_barrier()
        pltpu.sync_copy(acc_sh.at[pl.ds(row0, stripe)],
                        out_hbm.at[pl.ds(c * NPAD + row0, stripe)])

    return k


def _deg_call(NPAD, F, NCH):
    mesh = plsc.VectorSubcoreMesh(core_axis_name="c", subcore_axis_name="s", num_cores=NC, num_subcores=NS)
    stripe = NPAD // NS

    @functools.partial(
        pl.kernel,
        out_type=jax.ShapeDtypeStruct((NC * NPAD, F), jnp.float32),
        mesh=mesh,
        scratch_types=[
            pltpu.VMEM((NCH, CH), jnp.int32),
            pltpu.VMEM((CH, F), jnp.float32),
            pltpu.VMEM_SHARED((NPAD, F), jnp.float32),
        ],
    )
    def k(dst_hbm, zeros_hbm, ones_hbm, out_hbm, idx_d, ones_v, acc_sh):
        c = lax.axis_index("c")
        s = lax.axis_index("s")
        wid = c * NS + s
        row0 = s * stripe
        pltpu.sync_copy(dst_hbm.at[pl.ds(wid * NCH, NCH)], idx_d)
        pltpu.sync_copy(zeros_hbm.at[pl.ds(row0, stripe)],
                        acc_sh.at[pl.ds(row0, stripe)])
        pltpu.sync_copy(ones_hbm, ones_v)
        plsc.subcore_barrier()

        def body(kk, carry):
            pltpu.sync_copy(ones_v, acc_sh.at[idx_d.at[kk]], add=True)
            return carry

        lax.fori_loop(0, NCH, body, 0)
        plsc.subcore_barrier()
        pltpu.sync_copy(acc_sh.at[pl.ds(row0, stripe)],
                        out_hbm.at[pl.ds(c * NPAD + row0, stripe)])

    return k


def _first_call(NPAD, N, Fin, F):
    def body(x_ref, w_ref, degc_ref, s_ref, dinv_ref):
        deg = (degc_ref[0:NPAD, 0:1] + degc_ref[NPAD:2 * NPAD, 0:1]) + 1.0
        dinv = lax.rsqrt(jnp.maximum(deg, 1.0))
        dinv_ref[...] = dinv
        u = jnp.dot(x_ref[...], w_ref[...], preferred_element_type=jnp.float32)
        s_ref[0:N, :] = u * dinv[0:N]
        s_ref[N:NPAD, :] = jnp.zeros((NPAD - N, F), jnp.float32)

    return pl.pallas_call(
        body,
        out_shape=(
            jax.ShapeDtypeStruct((NPAD, F), jnp.float32),
            jax.ShapeDtypeStruct((NPAD, 1), jnp.float32),
        ),
    )


def _mid_call(NPAD, F):
    def body(agg_ref, sp_ref, dinv_ref, b_ref, w_ref, out_ref):
        a = agg_ref[0:NPAD, :] + agg_ref[NPAD:2 * NPAD, :] + sp_ref[...]
        dinv = dinv_ref[...]
        h = jnp.maximum(dinv * a + b_ref[...], 0.0)
        out_ref[...] = jnp.dot(h, w_ref[...],
                               preferred_element_type=jnp.float32) * dinv

    return pl.pallas_call(
        body, out_shape=jax.ShapeDtypeStruct((NPAD, F), jnp.float32))


def _pool_call(NPAD, N, F, NCLS):
    def body(agg_ref, sp_ref, dinv_ref, b_ref, batch_ref, lw_ref, lb_ref,
             out_ref):
        a = (agg_ref[0:N, :] + agg_ref[NPAD:NPAD + N, :] + sp_ref[0:N, :])
        h = jnp.maximum(dinv_ref[0:N, :] * a + b_ref[...], 0.0)
        gid = batch_ref[...]
        onehot = (gid == lax.broadcasted_iota(jnp.int32, (N, 128), 1)
                  ).astype(jnp.float32)
        sums = lax.dot_general(onehot, h, (((0,), (0,)), ((), ())),
                               preferred_element_type=jnp.float32)
        cnt = lax.dot_general(onehot, jnp.ones((N, 8), jnp.float32),
                              (((0,), (0,)), ((), ())),
                              preferred_element_type=jnp.float32)
        mean = sums / jnp.maximum(cnt[:, 0:1], 1.0)
        out_ref[...] = jnp.dot(mean, lw_ref[...],
                               preferred_element_type=jnp.float32) + lb_ref[...]

    return pl.pallas_call(
        body, out_shape=jax.ShapeDtypeStruct((128, NCLS), jnp.float32))


def kernel(x, edge_index, batch, W0, W1, W2, W3, W4, b0, b1, b2, b3, b4,
           lin_w, lin_b):
    N, Fin = x.shape
    F = W0.shape[1]
    E = edge_index.shape[1]
    NCLS = lin_w.shape[1]
    G = 64

    NPAD = ((N + 255) // 256) * 256
    NCH = -(-E // (NW * CH))
    NCH = NCH + (NCH % 2)
    EPAD = NW * CH * NCH

    pad_idx = (N + jnp.arange(EPAD - E, dtype=jnp.int32) % (NPAD - N))
    src = jnp.concatenate([edge_index[0], pad_idx]).reshape(NW * NCH, CH)
    dst = jnp.concatenate([edge_index[1], pad_idx]).reshape(NW * NCH, CH)
    zerosF = jnp.zeros((NPAD, F), jnp.float32)
    onesF = jnp.ones((CH, F), jnp.float32)
    Ws = [W0, W1, W2, W3, W4]
    bs = [b.reshape(1, F) for b in (b0, b1, b2, b3, b4)]

    agg = _agg_call(NPAD, F, NCH)
    degc = _deg_call(NPAD, F, NCH)(dst, zerosF, onesF)
    s, dinv = _first_call(NPAD, N, Fin, F)(x, W0, degc)
    for i in range(1, 5):
        a = agg(src, dst, s, zerosF)
        s = _mid_call(NPAD, F)(a, s, dinv, bs[i - 1], Ws[i])
    a = agg(src, dst, s, zerosF)
    out = _pool_call(NPAD, N, F, NCLS)(a, s, dinv, bs[4],
                                       batch.reshape(N, 1),
                                       lin_w, lin_b.reshape(1, NCLS))
    return out[:G]
